# Initial kernel scaffold; baseline (speedup 1.0000x reference)
import jax
import jax.numpy as jnp
from jax.experimental import pallas as pl


def kernel(x_atom, pos_atom, x_global_node, pos_global_node, edge_index_atom_atom, edge_index_atom_global_node, edge_index_global_node_atom, params):
    raise NotImplementedError("write your pallas kernel here")


# speedup vs baseline: 2.9230x; 2.9230x over previous
import functools

import jax
import jax.numpy as jnp
from jax import lax
from jax.experimental import pallas as pl
from jax.experimental.pallas import tpu as pltpu
from jax.experimental.pallas import tpu_sc as plsc

F = 128
NC = 2
NS = 16
NW = NC * NS
C = 80

f32 = jnp.float32


def _silu(x):
    return x * jax.nn.sigmoid(x)


def _ln(x, g, b, eps=1e-5):
    m = jnp.mean(x, axis=-1, keepdims=True)
    v = jnp.mean((x - m) ** 2, axis=-1, keepdims=True)
    return (x - m) / jnp.sqrt(v + eps) * g + b


def _pre_body(x_ref, wi_ref, wj_ref, xi_ref, xj_ref):
    x = x_ref[...]
    xi_ref[...] = jnp.dot(x, wi_ref[...], preferred_element_type=f32)
    xj_ref[...] = jnp.dot(x, wj_ref[...], preferred_element_type=f32)


def _pre_call(x, wi, wj, blk=2000):
    n = x.shape[0]
    return pl.pallas_call(
        _pre_body,
        grid=(n // blk,),
        in_specs=[pl.BlockSpec((blk, F), lambda i: (i, 0)),
                  pl.BlockSpec((F, F), lambda i: (0, 0)),
                  pl.BlockSpec((F, F), lambda i: (0, 0))],
        out_specs=[pl.BlockSpec((blk, F), lambda i: (i, 0))] * 2,
        out_shape=[jax.ShapeDtypeStruct((n, F), f32)] * 2,
    )(x, wi, wj)


def _sc_gather_call(xi, xj, px, py, pz, col, row):
    e = col.shape[0]
    n = xi.shape[0]
    ew = e // NW
    nch = ew // C
    g16 = ew // 16
    mesh = plsc.VectorSubcoreMesh(core_axis_name="c", subcore_axis_name="s")

    @functools.partial(
        pl.kernel,
        out_type=[jax.ShapeDtypeStruct((e, F), f32),
                  jax.ShapeDtypeStruct((e, F), f32),
                  jax.ShapeDtypeStruct((e,), f32),
                  jax.ShapeDtypeStruct((e,), f32),
                  jax.ShapeDtypeStruct((e,), f32)],
        mesh=mesh,
        scratch_types=[pltpu.VMEM((ew,), jnp.int32),
                       pltpu.VMEM((ew,), jnp.int32),
                       pltpu.VMEM((n,), f32),
                       pltpu.VMEM((n,), f32),
                       pltpu.VMEM((n,), f32),
                       pltpu.VMEM((ew,), f32),
                       pltpu.VMEM((ew,), f32),
                       pltpu.VMEM((ew,), f32),
                       pltpu.VMEM((C, F), f32),
                       pltpu.VMEM((C, F), f32),
                       pltpu.SemaphoreType.DMA,
                       pltpu.SemaphoreType.DMA],
        compiler_params=pltpu.CompilerParams(needs_layout_passes=False),
    )
    def k(xi_h, xj_h, px_h, py_h, pz_h, col_h, row_h,
          gi_h, gj_h, dx_h, dy_h, dz_h,
          colb, rowb, pxv, pyv, pzv, dxb, dyb, dzb, bufi, bufj, semi, semj):
        wid = lax.axis_index("s") * NC + lax.axis_index("c")
        base = wid * ew
        pltpu.sync_copy(col_h.at[pl.ds(base, ew)], colb)
        pltpu.sync_copy(row_h.at[pl.ds(base, ew)], rowb)
        pltpu.sync_copy(px_h, pxv)
        pltpu.sync_copy(py_h, pyv)
        pltpu.sync_copy(pz_h, pzv)

        def dbody(j, carry):
            sl = pl.ds(j * 16, 16)
            cv = colb[sl]
            rv = rowb[sl]
            dxb[sl] = plsc.load_gather(pxv, [cv]) - plsc.load_gather(pxv, [rv])
            dyb[sl] = plsc.load_gather(pyv, [cv]) - plsc.load_gather(pyv, [rv])
            dzb[sl] = plsc.load_gather(pzv, [cv]) - plsc.load_gather(pzv, [rv])
            return carry

        lax.fori_loop(0, g16, dbody, 0)
        pltpu.sync_copy(dxb, dx_h.at[pl.ds(base, ew)])
        pltpu.sync_copy(dyb, dy_h.at[pl.ds(base, ew)])
        pltpu.sync_copy(dzb, dz_h.at[pl.ds(base, ew)])

        def gbody(i, carry):
            off = i * C
            pltpu.async_copy(xi_h.at[colb.at[pl.ds(off, C)]], bufi, semi).wait()
            pltpu.sync_copy(bufi, gi_h.at[pl.ds(base + off, C)])
            pltpu.async_copy(xj_h.at[rowb.at[pl.ds(off, C)]], bufj, semj).wait()
            pltpu.sync_copy(bufj, gj_h.at[pl.ds(base + off, C)])
            return carry

        lax.fori_loop(0, nch, gbody, 0)

    return k(xi, xj, px, py, pz, col, row)


def _sc_scatter_call(h, pmx, pmy, pmz, col3, n):
    e = h.shape[0]
    ew = e // NW
    nch = col3.shape[1]
    nseg = (n // NS) // 8 * 8
    ntail = n - NS * nseg
    sb = max(d for d in range(8, 129, 8) if nseg % d == 0)
    reps = nseg // sb
    mesh = plsc.VectorSubcoreMesh(core_axis_name="c", subcore_axis_name="s")

    @functools.partial(
        pl.kernel,
        out_type=[jax.ShapeDtypeStruct((NC, n, F), f32),
                  jax.ShapeDtypeStruct((NC * n,), f32),
                  jax.ShapeDtypeStruct((NC * n,), f32),
                  jax.ShapeDtypeStruct((NC * n,), f32),
                  jax.ShapeDtypeStruct((NC * n,), f32)],
        mesh=mesh,
        scratch_types=[pltpu.VMEM((nch, C), jnp.int32),
                       pltpu.VMEM((C, F), f32),
                       pltpu.VMEM((C,), f32),
                       pltpu.VMEM((C,), f32),
                       pltpu.VMEM((C,), f32),
                       pltpu.VMEM((C,), f32),
                       pltpu.VMEM((sb, F), f32),
                       pltpu.VMEM((nseg,), f32),
                       pltpu.VMEM_SHARED((n, F), f32),
                       pltpu.VMEM_SHARED((n,), f32),
                       pltpu.VMEM_SHARED((n,), f32),
                       pltpu.VMEM_SHARED((n,), f32),
                       pltpu.VMEM_SHARED((n,), f32)],
        compiler_params=pltpu.CompilerParams(needs_layout_passes=False),
    )
    def k(h_h, pmx_h, pmy_h, pmz_h, col3_h,
          aggh_h, cnt_h, ax_h, ay_h, az_h,
          idxb, hbuf, pxb, pyb, pzb, onesb, sbuf, pbuf,
          acch, accc, accx, accy, accz):
        c = lax.axis_index("c")
        s = lax.axis_index("s")
        wid = s * NC + c
        base = wid * ew

        def zb(j, carry):
            sbuf[j // 8, pl.ds((j % 8) * 16, 16)] = jnp.zeros((16,), f32)
            return carry

        lax.fori_loop(0, sb * 8, zb, 0)

        def zp(j, carry):
            pbuf[pl.ds(j * 16, 16)] = jnp.zeros((16,), f32)
            return carry

        lax.fori_loop(0, nseg // 16, zp, 0)

        def ob(j, carry):
            onesb[pl.ds(j * 16, 16)] = jnp.ones((16,), f32)
            return carry

        lax.fori_loop(0, C // 16, ob, 0)

        for k in range(reps):
            pltpu.sync_copy(sbuf, acch.at[pl.ds(s * nseg + k * sb, sb)])
        segsl = pl.ds(s * nseg, nseg)
        pltpu.sync_copy(pbuf, accc.at[segsl])
        pltpu.sync_copy(pbuf, accx.at[segsl])
        pltpu.sync_copy(pbuf, accy.at[segsl])
        pltpu.sync_copy(pbuf, accz.at[segsl])

        @pl.when(s == 0)
        def _():
            tl = pl.ds(NS * nseg, ntail)
            tb = pl.ds(0, ntail)
            pltpu.sync_copy(sbuf.at[tb], acch.at[tl])
            pltpu.sync_copy(pbuf.at[tb], accc.at[tl])
            pltpu.sync_copy(pbuf.at[tb], accx.at[tl])
            pltpu.sync_copy(pbuf.at[tb], accy.at[tl])
            pltpu.sync_copy(pbuf.at[tb], accz.at[tl])

        pltpu.sync_copy(col3_h.at[wid], idxb)
        plsc.subcore_barrier()

        def body(j, carry):
            off = base + j * C
            pltpu.sync_copy(h_h.at[pl.ds(off, C)], hbuf)
            pltpu.sync_copy(pmx_h.at[pl.ds(off, C)], pxb)
            pltpu.sync_copy(pmy_h.at[pl.ds(off, C)], pyb)
            pltpu.sync_copy(pmz_h.at[pl.ds(off, C)], pzb)
            isl = idxb.at[j]
            pltpu.sync_copy(hbuf, acch.at[isl], add=True)
            pltpu.sync_copy(pxb, accx.at[isl], add=True)
            pltpu.sync_copy(pyb, accy.at[isl], add=True)
            pltpu.sync_copy(pzb, accz.at[isl], add=True)
            pltpu.sync_copy(onesb, accc.at[isl], add=True)
            return carry

        lax.fori_loop(0, nch, body, 0)
        plsc.subcore_barrier()

        for k in range(reps):
            ksl = pl.ds(s * nseg + k * sb, sb)
            pltpu.sync_copy(acch.at[ksl], sbuf)
            pltpu.sync_copy(sbuf, aggh_h.at[c, ksl])

        def dump_plane(acc, out_h):
            pltpu.sync_copy(acc.at[segsl], pbuf)
            pltpu.sync_copy(pbuf, out_h.at[pl.ds(c * n + s * nseg, nseg)])

        dump_plane(accc, cnt_h)
        dump_plane(accx, ax_h)
        dump_plane(accy, ay_h)
        dump_plane(accz, az_h)

        @pl.when(s == 0)
        def _():
            tl = pl.ds(NS * nseg, ntail)
            tb = pl.ds(0, ntail)
            pltpu.sync_copy(acch.at[tl], sbuf.at[tb])
            pltpu.sync_copy(sbuf.at[tb], aggh_h.at[c, tl])

            def dump_ptail(acc, out_h):
                pltpu.sync_copy(acc.at[tl], pbuf.at[tb])
                pltpu.sync_copy(pbuf.at[tb], out_h.at[pl.ds(c * n + NS * nseg, ntail)])

            dump_ptail(accc, cnt_h)
            dump_ptail(accx, ax_h)
            dump_ptail(accy, ay_h)
            dump_ptail(accz, az_h)

    return k(h, pmx, pmy, pmz, col3)


def _edge_body(gi_ref, gj_ref, dx_ref, dy_ref, dz_ref,
               wd_ref, b1_ref, w2_ref, b2_ref, wa_ref, ba_ref,
               p1_ref, d1_ref, p2_ref,
               h_ref, px_ref, py_ref, pz_ref):
    dx = dx_ref[...]
    dy = dy_ref[...]
    dz = dz_ref[...]
    dist = jnp.sqrt(dx * dx + dy * dy + dz * dz)
    pre = gi_ref[...] + gj_ref[...] + dist * wd_ref[...] + b1_ref[...]
    h1 = _silu(pre)
    h2 = _silu(jnp.dot(h1, w2_ref[...], preferred_element_type=f32) + b2_ref[...])
    att = jax.nn.sigmoid(jnp.dot(h2, wa_ref[...], preferred_element_type=f32)
                         + ba_ref[...])
    hh = att * h2
    pw = jnp.dot(_silu(jnp.dot(hh, p1_ref[...], preferred_element_type=f32)
                       + d1_ref[...]),
                 p2_ref[...], preferred_element_type=f32)
    h_ref[...] = hh
    px_ref[...] = dx * pw
    py_ref[...] = dy * pw
    pz_ref[...] = dz * pw


def _edge_call(gi, gj, dx, dy, dz, wd, b1, w2, b2, wa, ba, p1, d1, p2,
               blk=1280):
    e = gi.shape[0]
    full = lambda shp: pl.BlockSpec(shp, lambda i: tuple(0 for _ in shp))
    ef = pl.BlockSpec((blk, F), lambda i: (i, 0))
    e1 = pl.BlockSpec((blk, 1), lambda i: (i, 0))
    return pl.pallas_call(
        _edge_body,
        grid=(e // blk,),
        in_specs=[ef, ef, e1, e1, e1,
                  full((1, F)), full((1, F)), full((F, F)), full((1, F)),
                  full((F, 1)), full((1, 1)),
                  full((F, F)), full((1, F)), full((F, 1))],
        out_specs=[ef, e1, e1, e1],
        out_shape=[jax.ShapeDtypeStruct((e, F), f32),
                   jax.ShapeDtypeStruct((e, 1), f32),
                   jax.ShapeDtypeStruct((e, 1), f32),
                   jax.ShapeDtypeStruct((e, 1), f32)],
    )(gi, gj, dx, dy, dz, wd, b1, w2, b2, wa, ba, p1, d1, p2)


def _upd_body(x_ref, aggh_ref, cnt_ref, ax_ref, ay_ref, az_ref,
              px_ref, py_ref, pz_ref,
              u1x_ref, u1a_ref, c1_ref, u2_ref, c2_ref,
              xo_ref, pxo_ref, pyo_ref, pzo_ref):
    x = x_ref[...]
    aggh = aggh_ref[...]
    agg = aggh[0] + aggh[1]
    cntp = cnt_ref[...]
    cnt = jnp.clip(cntp[0] + cntp[1], 1.0, None)
    u = _silu(jnp.dot(x, u1x_ref[...], preferred_element_type=f32)
              + jnp.dot(agg, u1a_ref[...], preferred_element_type=f32)
              + c1_ref[...])
    xo_ref[...] = jnp.dot(u, u2_ref[...], preferred_element_type=f32) \
        + c2_ref[...] + x
    axp = ax_ref[...]
    ayp = ay_ref[...]
    azp = az_ref[...]
    pxo_ref[...] = px_ref[...] + (axp[0] + axp[1]) / cnt
    pyo_ref[...] = py_ref[...] + (ayp[0] + ayp[1]) / cnt
    pzo_ref[...] = pz_ref[...] + (azp[0] + azp[1]) / cnt


def _upd_call(x, aggh, cnt, ax, ay, az, px, py, pz,
              u1x, u1a, c1, u2, c2, blk=2000):
    n = x.shape[0]
    full = lambda shp: pl.BlockSpec(shp, lambda i: tuple(0 for _ in shp))
    nf = pl.BlockSpec((blk, F), lambda i: (i, 0))
    n1 = pl.BlockSpec((blk, 1), lambda i: (i, 0))
    pf = pl.BlockSpec((NC, blk, F), lambda i: (0, i, 0))
    p1s = pl.BlockSpec((NC, blk, 1), lambda i: (0, i, 0))
    return pl.pallas_call(
        _upd_body,
        grid=(n // blk,),
        in_specs=[nf, pf, p1s, p1s, p1s, p1s, n1, n1, n1,
                  full((F, F)), full((F, F)), full((1, F)),
                  full((F, F)), full((1, F))],
        out_specs=[nf, n1, n1, n1],
        out_shape=[jax.ShapeDtypeStruct((n, F), f32),
                   jax.ShapeDtypeStruct((n, 1), f32),
                   jax.ShapeDtypeStruct((n, 1), f32),
                   jax.ShapeDtypeStruct((n, 1), f32)],
    )(x, aggh, cnt, ax, ay, az, px, py, pz, u1x, u1a, c1, u2, c2)


def _a2g_body(x_ref, px_ref, py_ref, pz_ref, xg_ref, pg_ref,
              lng_ref, lnb_ref, wi_ref, wj_ref, wd_ref, b1_ref,
              w2_ref, b2_ref, p1_ref, d1_ref, p2_ref, cs_ref,
              hs_ref, psx_ref, psy_ref, psz_ref):
    xg_ln = _ln(xg_ref[...], lng_ref[...], lnb_ref[...])
    basei = jnp.dot(xg_ln, wi_ref[...], preferred_element_type=f32)
    xj = _ln(x_ref[...], lng_ref[...], lnb_ref[...])
    pg = pg_ref[...]
    dx = pg[0, 0] - px_ref[...]
    dy = pg[0, 1] - py_ref[...]
    dz = pg[0, 2] - pz_ref[...]
    dist = jnp.sqrt(dx * dx + dy * dy + dz * dz)
    h1 = _silu(basei + jnp.dot(xj, wj_ref[...], preferred_element_type=f32)
               + dist * wd_ref[...] + b1_ref[...])
    h2 = _silu(jnp.dot(h1, w2_ref[...], preferred_element_type=f32)
               + b2_ref[...])
    scale = cs_ref[0, 0] / jnp.clip(dist, 1e-8, None)
    pw = jnp.dot(_silu(jnp.dot(h2, p1_ref[...], preferred_element_type=f32)
                       + d1_ref[...]),
                 p2_ref[...], preferred_element_type=f32) * scale
    hs_ref[...] = jnp.sum(h2, axis=0, keepdims=True)[None]
    psx_ref[...] = jnp.sum(dx * pw, axis=0, keepdims=True)[None]
    psy_ref[...] = jnp.sum(dy * pw, axis=0, keepdims=True)[None]
    psz_ref[...] = jnp.sum(dz * pw, axis=0, keepdims=True)[None]


def _a2g_call(x, px, py, pz, xg, pg, p, blk=2000):
    n = x.shape[0]
    nb = n // blk
    full = lambda shp: pl.BlockSpec(shp, lambda i: tuple(0 for _ in shp))
    nf = pl.BlockSpec((blk, F), lambda i: (i, 0))
    n1 = pl.BlockSpec((blk, 1), lambda i: (i, 0))
    return pl.pallas_call(
        _a2g_body,
        grid=(nb,),
        in_specs=[nf, n1, n1, n1, full((1, F)), full((1, 3)),
                  full((1, F)), full((1, F)), full((F, F)), full((F, F)),
                  full((1, F)), full((1, F)), full((F, F)), full((1, F)),
                  full((F, F)), full((1, F)), full((F, 1)), full((1, 1))],
        out_specs=[pl.BlockSpec((1, 1, F), lambda i: (i, 0, 0)),
                   pl.BlockSpec((1, 1, 1), lambda i: (i, 0, 0)),
                   pl.BlockSpec((1, 1, 1), lambda i: (i, 0, 0)),
                   pl.BlockSpec((1, 1, 1), lambda i: (i, 0, 0))],
        out_shape=[jax.ShapeDtypeStruct((nb, 1, F), f32),
                   jax.ShapeDtypeStruct((nb, 1, 1), f32),
                   jax.ShapeDtypeStruct((nb, 1, 1), f32),
                   jax.ShapeDtypeStruct((nb, 1, 1), f32)],
    )(x, px, py, pz, xg, pg,
      p['ln_g'].reshape(1, F), p['ln_b'].reshape(1, F),
      p['W1'][:F], p['W1'][F:2 * F], p['W1'][2 * F:].reshape(1, F),
      p['b1'].reshape(1, F), p['W2'], p['b2'].reshape(1, F),
      p['P1'], p['d1'].reshape(1, F), p['P2'],
      p['coors_scale'].reshape(1, 1))


def _g2a_body(x_ref, px_ref, py_ref, pz_ref, xg_ref, pg_ref,
              hs_ref, psx_ref, psy_ref, psz_ref,
              gu1x_ref, gu1a_ref, gc1_ref, gu2_ref, gc2_ref,
              lng_ref, lnb_ref, wi_ref, wj_ref, wd_ref, b1_ref,
              w2_ref, b2_ref, p1_ref, d1_ref, p2_ref, cs_ref,
              u1x_ref, u1a_ref, c1_ref, u2_ref, c2_ref,
              xo_ref, pxo_ref, pyo_ref, pzo_ref,
              xgo_ref, pgo_ref, *, n_atoms):
    i = pl.program_id(0)
    xg = xg_ref[...]
    agg_g = jnp.sum(hs_ref[...], axis=0, keepdims=True)
    inv = 1.0 / n_atoms
    apx = jnp.sum(psx_ref[...]) * inv
    apy = jnp.sum(psy_ref[...]) * inv
    apz = jnp.sum(psz_ref[...]) * inv
    ug = _silu(jnp.dot(xg, gu1x_ref[...], preferred_element_type=f32)
               + jnp.dot(agg_g, gu1a_ref[...], preferred_element_type=f32)
               + gc1_ref[...])
    xg_new = jnp.dot(ug, gu2_ref[...], preferred_element_type=f32) \
        + gc2_ref[...] + xg
    pg = pg_ref[...]
    pgx = pg[0, 0] + apx
    pgy = pg[0, 1] + apy
    pgz = pg[0, 2] + apz
    x = x_ref[...]
    xin_i = _ln(x, lng_ref[...], lnb_ref[...])
    xg_ln = _ln(xg_new, lng_ref[...], lnb_ref[...])
    basej = jnp.dot(xg_ln, wj_ref[...], preferred_element_type=f32)
    dx = px_ref[...] - pgx
    dy = py_ref[...] - pgy
    dz = pz_ref[...] - pgz
    dist = jnp.sqrt(dx * dx + dy * dy + dz * dz)
    h1 = _silu(jnp.dot(xin_i, wi_ref[...], preferred_element_type=f32)
               + basej + dist * wd_ref[...] + b1_ref[...])
    h2 = _silu(jnp.dot(h1, w2_ref[...], preferred_element_type=f32)
               + b2_ref[...])
    scale = cs_ref[0, 0] / jnp.clip(dist, 1e-8, None)
    pw = jnp.dot(_silu(jnp.dot(h2, p1_ref[...], preferred_element_type=f32)
                       + d1_ref[...]),
                 p2_ref[...], preferred_element_type=f32) * scale
    u = _silu(jnp.dot(x, u1x_ref[...], preferred_element_type=f32)
              + jnp.dot(h2, u1a_ref[...], preferred_element_type=f32)
              + c1_ref[...])
    xo_ref[...] = jnp.dot(u, u2_ref[...], preferred_element_type=f32) \
        + c2_ref[...] + x
    pxo_ref[...] = px_ref[...] + dx * pw
    pyo_ref[...] = py_ref[...] + dy * pw
    pzo_ref[...] = pz_ref[...] + dz * pw

    @pl.when(i == 0)
    def _():
        xgo_ref[...] = xg_new
        pgo_ref[...] = jnp.concatenate(
            [jnp.full((1, 1), pgx, f32), jnp.full((1, 1), pgy, f32),
             jnp.full((1, 1), pgz, f32)], axis=1)


def _g2a_call(x, px, py, pz, xg, pg, hs, psx, psy, psz, pg_upd, p,
              blk=2000):
    n = x.shape[0]
    nb8 = hs.shape[0]
    full = lambda shp: pl.BlockSpec(shp, lambda i: tuple(0 for _ in shp))
    nf = pl.BlockSpec((blk, F), lambda i: (i, 0))
    n1 = pl.BlockSpec((blk, 1), lambda i: (i, 0))
    body = functools.partial(_g2a_body, n_atoms=float(n))
    return pl.pallas_call(
        body,
        grid=(n // blk,),
        in_specs=[nf, n1, n1, n1, full((1, F)), full((1, 3)),
                  full((nb8, F)), full((nb8, 1)), full((nb8, 1)),
                  full((nb8, 1)),
                  full((F, F)), full((F, F)), full((1, F)), full((F, F)),
                  full((1, F)),
                  full((1, F)), full((1, F)), full((F, F)), full((F, F)),
                  full((1, F)), full((1, F)), full((F, F)), full((1, F)),
                  full((F, F)), full((1, F)), full((F, 1)), full((1, 1)),
                  full((F, F)), full((F, F)), full((1, F)), full((F, F)),
                  full((1, F))],
        out_specs=[nf, n1, n1, n1,
                   pl.BlockSpec((1, F), lambda i: (0, 0)),
                   pl.BlockSpec((1, 3), lambda i: (0, 0))],
        out_shape=[jax.ShapeDtypeStruct((n, F), f32),
                   jax.ShapeDtypeStruct((n, 1), f32),
                   jax.ShapeDtypeStruct((n, 1), f32),
                   jax.ShapeDtypeStruct((n, 1), f32),
                   jax.ShapeDtypeStruct((1, F), f32),
                   jax.ShapeDtypeStruct((1, 3), f32)],
    )(x, px, py, pz, xg, pg, hs, psx, psy, psz,
      pg_upd['U1'][:F], pg_upd['U1'][F:], pg_upd['c1'].reshape(1, F),
      pg_upd['U2'], pg_upd['c2'].reshape(1, F),
      p['ln_g'].reshape(1, F), p['ln_b'].reshape(1, F),
      p['W1'][:F], p['W1'][F:2 * F], p['W1'][2 * F:].reshape(1, F),
      p['b1'].reshape(1, F), p['W2'], p['b2'].reshape(1, F),
      p['P1'], p['d1'].reshape(1, F), p['P2'],
      p['coors_scale'].reshape(1, 1),
      p['U1'][:F], p['U1'][F:], p['c1'].reshape(1, F),
      p['U2'], p['c2'].reshape(1, F))


def kernel(x_atom, pos_atom, x_global_node, pos_global_node,
           edge_index_atom_atom, edge_index_atom_global_node,
           edge_index_global_node_atom, params):
    n = x_atom.shape[0]
    e = edge_index_atom_atom.shape[1]
    row = edge_index_atom_atom[0]
    col = edge_index_atom_atom[1]
    ew = e // NW
    col3 = col.reshape(NW, ew // C, C)

    x = x_atom
    px = pos_atom[:, 0:1]
    py = pos_atom[:, 1:2]
    pz = pos_atom[:, 2:3]
    xg = x_global_node
    pg = pos_global_node

    for l in range(len(params['layers'])):
        pa = params['layers'][l]['a2a']
        pag = params['layers'][l]['a2g']
        pga = params['layers'][l]['g2a']

        xi, xj = _pre_call(x, pa['W1'][:F], pa['W1'][F:2 * F])
        gi, gj, dx, dy, dz = _sc_gather_call(
            xi, xj, px.reshape(n), py.reshape(n), pz.reshape(n), col, row)
        h, pmx, pmy, pmz = _edge_call(
            gi, gj, dx.reshape(e, 1), dy.reshape(e, 1), dz.reshape(e, 1),
            pa['W1'][2 * F:].reshape(1, F), pa['b1'].reshape(1, F),
            pa['W2'], pa['b2'].reshape(1, F),
            pa['Wa'], pa['ba'].reshape(1, 1),
            pa['P1'], pa['d1'].reshape(1, F), pa['P2'])
        aggh, cnt, ax, ay, az = _sc_scatter_call(
            h, pmx.reshape(e), pmy.reshape(e), pmz.reshape(e),
            col3, n)
        x, px, py, pz = _upd_call(
            x, aggh, cnt.reshape(NC, n, 1), ax.reshape(NC, n, 1),
            ay.reshape(NC, n, 1), az.reshape(NC, n, 1), px, py, pz,
            pa['U1'][:F], pa['U1'][F:], pa['c1'].reshape(1, F),
            pa['U2'], pa['c2'].reshape(1, F))

        hs, psx, psy, psz = _a2g_call(x, px, py, pz, xg, pg, pag)
        nb = hs.shape[0]
        hs = hs.reshape(nb, F)
        psx = psx.reshape(nb, 1)
        psy = psy.reshape(nb, 1)
        psz = psz.reshape(nb, 1)
        pad = (-nb) % 8
        if pad:
            hs = jnp.pad(hs, ((0, pad), (0, 0)))
            psx = jnp.pad(psx, ((0, pad), (0, 0)))
            psy = jnp.pad(psy, ((0, pad), (0, 0)))
            psz = jnp.pad(psz, ((0, pad), (0, 0)))

        x, px, py, pz, xg, pg = _g2a_call(
            x, px, py, pz, xg, pg, hs, psx, psy, psz, pag, pga)

    pos_atom_out = jnp.concatenate([px, py, pz], axis=1)
    return x, pos_atom_out, xg, pg


# speedup vs baseline: 3.6149x; 1.2367x over previous
import functools

import jax
import jax.numpy as jnp
from jax import lax
from jax.experimental import pallas as pl
from jax.experimental.pallas import tpu as pltpu
from jax.experimental.pallas import tpu_sc as plsc

F = 128
NC = 2
NS = 16
NW = NC * NS
C = 80

f32 = jnp.float32


def _silu(x):
    return x * jax.nn.sigmoid(x)


def _ln(x, g, b, eps=1e-5):
    m = jnp.mean(x, axis=-1, keepdims=True)
    v = jnp.mean((x - m) ** 2, axis=-1, keepdims=True)
    return (x - m) / jnp.sqrt(v + eps) * g + b


def _pre_body(x_ref, wi_ref, wj_ref, xi_ref, xj_ref):
    x = x_ref[...]
    xi_ref[...] = jnp.dot(x, wi_ref[...], preferred_element_type=f32)
    xj_ref[...] = jnp.dot(x, wj_ref[...], preferred_element_type=f32)


def _pre_call(x, wi, wj, blk=2000):
    n = x.shape[0]
    return pl.pallas_call(
        _pre_body,
        grid=(n // blk,),
        in_specs=[pl.BlockSpec((blk, F), lambda i: (i, 0)),
                  pl.BlockSpec((F, F), lambda i: (0, 0)),
                  pl.BlockSpec((F, F), lambda i: (0, 0))],
        out_specs=[pl.BlockSpec((blk, F), lambda i: (i, 0))] * 2,
        out_shape=[jax.ShapeDtypeStruct((n, F), f32)] * 2,
    )(x, wi, wj)


SCE = 80


def _sc_gather_call(xi, xj, px, py, pz, col, row):
    e = col.shape[0]
    n = xi.shape[0]
    ew = e // NW
    nsc = ew // SCE
    mesh = plsc.VectorSubcoreMesh(core_axis_name="c", subcore_axis_name="s")

    @functools.partial(
        pl.kernel,
        out_type=[jax.ShapeDtypeStruct((e, F), f32),
                  jax.ShapeDtypeStruct((e, F), f32),
                  jax.ShapeDtypeStruct((e,), f32),
                  jax.ShapeDtypeStruct((e,), f32),
                  jax.ShapeDtypeStruct((e,), f32)],
        mesh=mesh,
        scratch_types=[pltpu.VMEM((SCE,), jnp.int32),
                       pltpu.VMEM((SCE,), jnp.int32),
                       pltpu.VMEM((SCE,), jnp.int32),
                       pltpu.VMEM((SCE,), jnp.int32),
                       pltpu.VMEM((n,), f32),
                       pltpu.VMEM((n,), f32),
                       pltpu.VMEM((n,), f32),
                       pltpu.VMEM((ew,), f32),
                       pltpu.VMEM((ew,), f32),
                       pltpu.VMEM((ew,), f32),
                       pltpu.VMEM((SCE, F), f32),
                       pltpu.VMEM((SCE, F), f32),
                       pltpu.VMEM((SCE, F), f32),
                       pltpu.VMEM((SCE, F), f32),
                       pltpu.SemaphoreType.DMA,
                       pltpu.SemaphoreType.DMA,
                       pltpu.SemaphoreType.DMA,
                       pltpu.SemaphoreType.DMA,
                       pltpu.SemaphoreType.DMA,
                       pltpu.SemaphoreType.DMA],
        compiler_params=pltpu.CompilerParams(needs_layout_passes=False),
    )
    def k(xi_h, xj_h, px_h, py_h, pz_h, col_h, row_h,
          gi_h, gj_h, dx_h, dy_h, dz_h,
          ica, icb, ira, irb, pxv, pyv, pzv, dbx, dby, dbz,
          obia, obib, obja, objb,
          sia, sib, sga, sgb, swa, swb):
        wid = lax.axis_index("s") * NC + lax.axis_index("c")
        base = wid * ew
        pltpu.sync_copy(px_h, pxv)
        pltpu.sync_copy(py_h, pyv)
        pltpu.sync_copy(pz_h, pzv)

        ic = (ica, icb)
        ir = (ira, irb)
        obi = (obia, obib)
        obj = (obja, objb)
        si = (sia, sib)
        sg = (sga, sgb)
        sw = (swa, swb)

        def idescs(t, p):
            sl = pl.ds(base + t * SCE, SCE)
            return [(col_h.at[sl], ic[p], si[p]),
                    (row_h.at[sl], ir[p], si[p])]

        def istart(t, p):
            for (src, dst, sem) in idescs(t, p):
                pltpu.async_copy(src, dst, sem)

        def iwait(t, p):
            for (src, dst, sem) in idescs(t, p):
                pltpu.make_async_copy(src, dst, sem).wait()

        def gdescs(t, p):
            return [(xi_h.at[ic[p]], obi[p], sg[p]),
                    (xj_h.at[ir[p]], obj[p], sg[p])]

        def gstart(t, p):
            for (src, dst, sem) in gdescs(t, p):
                pltpu.async_copy(src, dst, sem)

        def gwait(t, p):
            for (src, dst, sem) in gdescs(t, p):
                pltpu.make_async_copy(src, dst, sem).wait()

        def wdescs(t, p):
            osl = pl.ds(base + t * SCE, SCE)
            return [(obi[p], gi_h.at[osl], sw[p]),
                    (obj[p], gj_h.at[osl], sw[p])]

        def wstart(t, p):
            for (src, dst, sem) in wdescs(t, p):
                pltpu.async_copy(src, dst, sem)

        def wwait(t, p):
            for (src, dst, sem) in wdescs(t, p):
                pltpu.make_async_copy(src, dst, sem).wait()

        def dircompute(t, p):
            def g16(q, carry):
                sl = pl.ds(q * 16, 16)
                osl = pl.ds(t * SCE + q * 16, 16)
                cv = ic[p][sl]
                rv = ir[p][sl]
                dbx[osl] = (plsc.load_gather(pxv, [cv])
                            - plsc.load_gather(pxv, [rv]))
                dby[osl] = (plsc.load_gather(pyv, [cv])
                            - plsc.load_gather(pyv, [rv]))
                dbz[osl] = (plsc.load_gather(pzv, [cv])
                            - plsc.load_gather(pzv, [rv]))
                return carry

            lax.fori_loop(0, SCE // 16, g16, 0)

        istart(0, 0)
        iwait(0, 0)
        istart(1, 1)
        gstart(0, 0)
        dircompute(0, 0)
        gwait(0, 0)
        wstart(0, 0)

        def pair(tt, carry):
            t1 = 2 * tt + 1
            iwait(t1, 1)

            @pl.when(t1 >= 2)
            def _():
                wwait(t1 - 2, 1)

            gstart(t1, 1)
            istart(t1 + 1, 0)
            dircompute(t1, 1)
            gwait(t1, 1)
            wstart(t1, 1)

            t2 = 2 * tt + 2
            iwait(t2, 0)
            wwait(t2 - 2, 0)
            gstart(t2, 0)

            @pl.when(t2 + 1 < nsc)
            def _():
                istart(t2 + 1, 1)

            dircompute(t2, 0)
            gwait(t2, 0)
            wstart(t2, 0)
            return carry

        lax.fori_loop(0, (nsc - 1) // 2, pair, 0)
        wwait(nsc - 2, 1)
        wwait(nsc - 1, 0)
        pltpu.sync_copy(dbx, dx_h.at[pl.ds(base, ew)])
        pltpu.sync_copy(dby, dy_h.at[pl.ds(base, ew)])
        pltpu.sync_copy(dbz, dz_h.at[pl.ds(base, ew)])

    return k(xi, xj, px, py, pz, col, row)


def _sc_scatter_call(h, pmx, pmy, pmz, col3, n):
    e = h.shape[0]
    ew = e // NW
    nch = col3.shape[1]
    nseg = (n // NS) // 8 * 8
    ntail = n - NS * nseg
    sb = max(d for d in range(8, 49, 8) if nseg % d == 0)
    reps = nseg // sb
    mesh = plsc.VectorSubcoreMesh(core_axis_name="c", subcore_axis_name="s")

    @functools.partial(
        pl.kernel,
        out_type=[jax.ShapeDtypeStruct((NC, n, F), f32),
                  jax.ShapeDtypeStruct((NC * n,), f32),
                  jax.ShapeDtypeStruct((NC * n,), f32),
                  jax.ShapeDtypeStruct((NC * n,), f32),
                  jax.ShapeDtypeStruct((NC * n,), f32)],
        mesh=mesh,
        scratch_types=[pltpu.VMEM((nch, C), jnp.int32),
                       pltpu.VMEM((C,), f32),
                       pltpu.VMEM((C,), f32),
                       pltpu.VMEM((C,), f32),
                       pltpu.VMEM((C,), f32),
                       pltpu.VMEM((C,), f32),
                       pltpu.VMEM((C,), f32),
                       pltpu.VMEM((C,), f32),
                       pltpu.VMEM((C, F), f32),
                       pltpu.VMEM((C, F), f32),
                       pltpu.VMEM((sb, F), f32),
                       pltpu.VMEM((nseg,), f32),
                       pltpu.SemaphoreType.DMA,
                       pltpu.SemaphoreType.DMA,
                       pltpu.SemaphoreType.DMA,
                       pltpu.SemaphoreType.DMA,
                       pltpu.SemaphoreType.DMA,
                       pltpu.VMEM_SHARED((n, F), f32),
                       pltpu.VMEM_SHARED((n,), f32),
                       pltpu.VMEM_SHARED((n,), f32),
                       pltpu.VMEM_SHARED((n,), f32),
                       pltpu.VMEM_SHARED((n,), f32)],
        compiler_params=pltpu.CompilerParams(needs_layout_passes=False),
    )
    def k(h_h, pmx_h, pmy_h, pmz_h, col3_h,
          aggh_h, cnt_h, ax_h, ay_h, az_h,
          idxb, pxa, pxb2, pya, pyb2, pza, pzb2, onesb, hba, hbb, sbuf, pbuf,
          sha, shb, saa, sab, sp4,
          acch, accc, accx, accy, accz):
        c = lax.axis_index("c")
        s = lax.axis_index("s")
        wid = s * NC + c
        base = wid * ew
        hb = (hba, hbb)
        pxs = (pxa, pxb2)
        pys = (pya, pyb2)
        pzs = (pza, pzb2)
        sh = (sha, shb)
        sa = (saa, sab)

        def zb(j, carry):
            sbuf[j // 8, pl.ds((j % 8) * 16, 16)] = jnp.zeros((16,), f32)
            return carry

        lax.fori_loop(0, sb * 8, zb, 0)

        def zp(j, carry):
            pbuf[pl.ds(j * 16, 16)] = jnp.zeros((16,), f32)
            return carry

        lax.fori_loop(0, nseg // 16, zp, 0)

        def ob(j, carry):
            onesb[pl.ds(j * 16, 16)] = jnp.ones((16,), f32)
            return carry

        lax.fori_loop(0, C // 16, ob, 0)

        pltpu.sync_copy(col3_h.at[wid], idxb)

        for r in range(reps):
            pltpu.sync_copy(sbuf, acch.at[pl.ds(s * nseg + r * sb, sb)])
        segsl = pl.ds(s * nseg, nseg)
        pltpu.sync_copy(pbuf, accc.at[segsl])
        pltpu.sync_copy(pbuf, accx.at[segsl])
        pltpu.sync_copy(pbuf, accy.at[segsl])
        pltpu.sync_copy(pbuf, accz.at[segsl])

        @pl.when(s == 0)
        def _():
            tl = pl.ds(NS * nseg, ntail)
            tb = pl.ds(0, ntail)
            pltpu.sync_copy(sbuf.at[tb], acch.at[tl])
            pltpu.sync_copy(pbuf.at[tb], accc.at[tl])
            pltpu.sync_copy(pbuf.at[tb], accx.at[tl])
            pltpu.sync_copy(pbuf.at[tb], accy.at[tl])
            pltpu.sync_copy(pbuf.at[tb], accz.at[tl])

        plsc.subcore_barrier()

        def hdescs(t, p):
            sl = pl.ds(base + t * C, C)
            return [(h_h.at[sl], hb[p], sh[p]),
                    (pmx_h.at[sl], pxs[p], sh[p]),
                    (pmy_h.at[sl], pys[p], sh[p]),
                    (pmz_h.at[sl], pzs[p], sh[p])]

        def hstart(t, p):
            for (src, dst, sem) in hdescs(t, p):
                pltpu.async_copy(src, dst, sem)

        def hwait(t, p):
            for (src, dst, sem) in hdescs(t, p):
                pltpu.make_async_copy(src, dst, sem).wait()

        def adescs(t, p):
            isl = idxb.at[t]
            return [(hb[p], acch.at[isl], sa[p]),
                    (pxs[p], accx.at[isl], sa[p]),
                    (pys[p], accy.at[isl], sa[p]),
                    (pzs[p], accz.at[isl], sa[p])]

        def astart(t, p):
            for (src, dst, sem) in adescs(t, p):
                pltpu.async_copy(src, dst, sem, add=True)
            pltpu.async_copy(onesb, accc.at[idxb.at[t]], sp4, add=True)

        def adrain(t, p):
            for (src, dst, sem) in adescs(t, p):
                pltpu.make_async_copy(src, dst, sem).wait()

        hstart(0, 0)
        hwait(0, 0)
        hstart(1, 1)
        astart(0, 0)

        def pair(tt, carry):
            t1 = 2 * tt + 1
            hwait(t1, 1)
            adrain(t1 - 1, 0)
            hstart(t1 + 1, 0)
            astart(t1, 1)
            t2 = 2 * tt + 2
            hwait(t2, 0)
            adrain(t2 - 1, 1)

            @pl.when(t2 + 1 < nch)
            def _():
                hstart(t2 + 1, 1)

            astart(t2, 0)
            return carry

        lax.fori_loop(0, (nch - 1) // 2, pair, 0)
        adrain(nch - 1, 0)

        def p4drain(t, carry):
            pltpu.make_async_copy(onesb, accc.at[idxb.at[t]], sp4).wait()
            return carry

        lax.fori_loop(0, nch, p4drain, 0)
        plsc.subcore_barrier()

        for r in range(reps):
            ksl = pl.ds(s * nseg + r * sb, sb)
            pltpu.sync_copy(acch.at[ksl], sbuf)
            pltpu.sync_copy(sbuf, aggh_h.at[c, ksl])

        def dump_plane(acc, out_h):
            pltpu.sync_copy(acc.at[segsl], pbuf)
            pltpu.sync_copy(pbuf, out_h.at[pl.ds(c * n + s * nseg, nseg)])

        dump_plane(accc, cnt_h)
        dump_plane(accx, ax_h)
        dump_plane(accy, ay_h)
        dump_plane(accz, az_h)

        @pl.when(s == 0)
        def _():
            tl = pl.ds(NS * nseg, ntail)
            tb = pl.ds(0, ntail)
            pltpu.sync_copy(acch.at[tl], sbuf.at[tb])
            pltpu.sync_copy(sbuf.at[tb], aggh_h.at[c, tl])

            def dump_ptail(acc, out_h):
                pltpu.sync_copy(acc.at[tl], pbuf.at[tb])
                pltpu.sync_copy(pbuf.at[tb],
                                out_h.at[pl.ds(c * n + NS * nseg, ntail)])

            dump_ptail(accc, cnt_h)
            dump_ptail(accx, ax_h)
            dump_ptail(accy, ay_h)
            dump_ptail(accz, az_h)

    return k(h, pmx, pmy, pmz, col3)


def _edge_body(gi_ref, gj_ref, dx_ref, dy_ref, dz_ref,
               wd_ref, b1_ref, w2_ref, b2_ref, wa_ref, ba_ref,
               p1_ref, d1_ref, p2_ref,
               h_ref, pm4_ref):
    dx = dx_ref[...]
    dy = dy_ref[...]
    dz = dz_ref[...]
    dist = jnp.sqrt(dx * dx + dy * dy + dz * dz)
    pre = gi_ref[...] + gj_ref[...] + dist * wd_ref[...] + b1_ref[...]
    h1 = _silu(pre)
    h2 = _silu(jnp.dot(h1, w2_ref[...], preferred_element_type=f32) + b2_ref[...])
    att = jax.nn.sigmoid(jnp.dot(h2, wa_ref[...], preferred_element_type=f32)
                         + ba_ref[...])
    hh = att * h2
    pw = jnp.dot(_silu(jnp.dot(hh, p1_ref[...], preferred_element_type=f32)
                       + d1_ref[...]),
                 p2_ref[...], preferred_element_type=f32)
    h_ref[...] = hh
    pm4_ref[...] = jnp.concatenate(
        [dx * pw, dy * pw, dz * pw, jnp.ones_like(pw)], axis=1)


def _edge_call(gi, gj, dx, dy, dz, wd, b1, w2, b2, wa, ba, p1, d1, p2,
               blk=1280):
    e = gi.shape[0]
    full = lambda shp: pl.BlockSpec(shp, lambda i: tuple(0 for _ in shp))
    ef = pl.BlockSpec((blk, F), lambda i: (i, 0))
    e1 = pl.BlockSpec((blk, 1), lambda i: (i, 0))
    e4 = pl.BlockSpec((blk, 4), lambda i: (i, 0))
    return pl.pallas_call(
        _edge_body,
        grid=(e // blk,),
        in_specs=[ef, ef, e1, e1, e1,
                  full((1, F)), full((1, F)), full((F, F)), full((1, F)),
                  full((F, 1)), full((1, 1)),
                  full((F, F)), full((1, F)), full((F, 1))],
        out_specs=[ef, e4],
        out_shape=[jax.ShapeDtypeStruct((e, F), f32),
                   jax.ShapeDtypeStruct((e, 4), f32)],
    )(gi, gj, dx, dy, dz, wd, b1, w2, b2, wa, ba, p1, d1, p2)


def _upd_body(x_ref, aggh_ref, cnt_ref, ax_ref, ay_ref, az_ref,
              px_ref, py_ref, pz_ref,
              u1x_ref, u1a_ref, c1_ref, u2_ref, c2_ref,
              xo_ref, pxo_ref, pyo_ref, pzo_ref):
    x = x_ref[...]
    aggh = aggh_ref[...]
    agg = aggh[0] + aggh[1]
    cntp = cnt_ref[...]
    cnt = jnp.clip(cntp[0] + cntp[1], 1.0, None)
    u = _silu(jnp.dot(x, u1x_ref[...], preferred_element_type=f32)
              + jnp.dot(agg, u1a_ref[...], preferred_element_type=f32)
              + c1_ref[...])
    xo_ref[...] = jnp.dot(u, u2_ref[...], preferred_element_type=f32) \
        + c2_ref[...] + x
    axp = ax_ref[...]
    ayp = ay_ref[...]
    azp = az_ref[...]
    pxo_ref[...] = px_ref[...] + (axp[0] + axp[1]) / cnt
    pyo_ref[...] = py_ref[...] + (ayp[0] + ayp[1]) / cnt
    pzo_ref[...] = pz_ref[...] + (azp[0] + azp[1]) / cnt


def _upd_call(x, aggh, cnt, ax, ay, az, px, py, pz,
              u1x, u1a, c1, u2, c2, blk=2000):
    n = x.shape[0]
    full = lambda shp: pl.BlockSpec(shp, lambda i: tuple(0 for _ in shp))
    nf = pl.BlockSpec((blk, F), lambda i: (i, 0))
    n1 = pl.BlockSpec((blk, 1), lambda i: (i, 0))
    pf = pl.BlockSpec((NC, blk, F), lambda i: (0, i, 0))
    p1s = pl.BlockSpec((NC, blk, 1), lambda i: (0, i, 0))
    return pl.pallas_call(
        _upd_body,
        grid=(n // blk,),
        in_specs=[nf, pf, p1s, p1s, p1s, p1s, n1, n1, n1,
                  full((F, F)), full((F, F)), full((1, F)),
                  full((F, F)), full((1, F))],
        out_specs=[nf, n1, n1, n1],
        out_shape=[jax.ShapeDtypeStruct((n, F), f32),
                   jax.ShapeDtypeStruct((n, 1), f32),
                   jax.ShapeDtypeStruct((n, 1), f32),
                   jax.ShapeDtypeStruct((n, 1), f32)],
    )(x, aggh, cnt, ax, ay, az, px, py, pz, u1x, u1a, c1, u2, c2)


def _a2g_body(x_ref, px_ref, py_ref, pz_ref, xg_ref, pg_ref,
              lng_ref, lnb_ref, wi_ref, wj_ref, wd_ref, b1_ref,
              w2_ref, b2_ref, p1_ref, d1_ref, p2_ref, cs_ref,
              hs_ref, psx_ref, psy_ref, psz_ref):
    xg_ln = _ln(xg_ref[...], lng_ref[...], lnb_ref[...])
    basei = jnp.dot(xg_ln, wi_ref[...], preferred_element_type=f32)
    xj = _ln(x_ref[...], lng_ref[...], lnb_ref[...])
    pg = pg_ref[...]
    dx = pg[0, 0] - px_ref[...]
    dy = pg[0, 1] - py_ref[...]
    dz = pg[0, 2] - pz_ref[...]
    dist = jnp.sqrt(dx * dx + dy * dy + dz * dz)
    h1 = _silu(basei + jnp.dot(xj, wj_ref[...], preferred_element_type=f32)
               + dist * wd_ref[...] + b1_ref[...])
    h2 = _silu(jnp.dot(h1, w2_ref[...], preferred_element_type=f32)
               + b2_ref[...])
    scale = cs_ref[0, 0] / jnp.clip(dist, 1e-8, None)
    pw = jnp.dot(_silu(jnp.dot(h2, p1_ref[...], preferred_element_type=f32)
                       + d1_ref[...]),
                 p2_ref[...], preferred_element_type=f32) * scale
    hs_ref[...] = jnp.sum(h2, axis=0, keepdims=True)[None]
    psx_ref[...] = jnp.sum(dx * pw, axis=0, keepdims=True)[None]
    psy_ref[...] = jnp.sum(dy * pw, axis=0, keepdims=True)[None]
    psz_ref[...] = jnp.sum(dz * pw, axis=0, keepdims=True)[None]


def _a2g_call(x, px, py, pz, xg, pg, p, blk=2000):
    n = x.shape[0]
    nb = n // blk
    full = lambda shp: pl.BlockSpec(shp, lambda i: tuple(0 for _ in shp))
    nf = pl.BlockSpec((blk, F), lambda i: (i, 0))
    n1 = pl.BlockSpec((blk, 1), lambda i: (i, 0))
    return pl.pallas_call(
        _a2g_body,
        grid=(nb,),
        in_specs=[nf, n1, n1, n1, full((1, F)), full((1, 3)),
                  full((1, F)), full((1, F)), full((F, F)), full((F, F)),
                  full((1, F)), full((1, F)), full((F, F)), full((1, F)),
                  full((F, F)), full((1, F)), full((F, 1)), full((1, 1))],
        out_specs=[pl.BlockSpec((1, 1, F), lambda i: (i, 0, 0)),
                   pl.BlockSpec((1, 1, 1), lambda i: (i, 0, 0)),
                   pl.BlockSpec((1, 1, 1), lambda i: (i, 0, 0)),
                   pl.BlockSpec((1, 1, 1), lambda i: (i, 0, 0))],
        out_shape=[jax.ShapeDtypeStruct((nb, 1, F), f32),
                   jax.ShapeDtypeStruct((nb, 1, 1), f32),
                   jax.ShapeDtypeStruct((nb, 1, 1), f32),
                   jax.ShapeDtypeStruct((nb, 1, 1), f32)],
    )(x, px, py, pz, xg, pg,
      p['ln_g'].reshape(1, F), p['ln_b'].reshape(1, F),
      p['W1'][:F], p['W1'][F:2 * F], p['W1'][2 * F:].reshape(1, F),
      p['b1'].reshape(1, F), p['W2'], p['b2'].reshape(1, F),
      p['P1'], p['d1'].reshape(1, F), p['P2'],
      p['coors_scale'].reshape(1, 1))


def _g2a_body(x_ref, px_ref, py_ref, pz_ref, xg_ref, pg_ref,
              hs_ref, psx_ref, psy_ref, psz_ref,
              gu1x_ref, gu1a_ref, gc1_ref, gu2_ref, gc2_ref,
              lng_ref, lnb_ref, wi_ref, wj_ref, wd_ref, b1_ref,
              w2_ref, b2_ref, p1_ref, d1_ref, p2_ref, cs_ref,
              u1x_ref, u1a_ref, c1_ref, u2_ref, c2_ref,
              xo_ref, pxo_ref, pyo_ref, pzo_ref,
              xgo_ref, pgo_ref, *, n_atoms):
    i = pl.program_id(0)
    xg = xg_ref[...]
    agg_g = jnp.sum(hs_ref[...], axis=0, keepdims=True)
    inv = 1.0 / n_atoms
    apx = jnp.sum(psx_ref[...]) * inv
    apy = jnp.sum(psy_ref[...]) * inv
    apz = jnp.sum(psz_ref[...]) * inv
    ug = _silu(jnp.dot(xg, gu1x_ref[...], preferred_element_type=f32)
               + jnp.dot(agg_g, gu1a_ref[...], preferred_element_type=f32)
               + gc1_ref[...])
    xg_new = jnp.dot(ug, gu2_ref[...], preferred_element_type=f32) \
        + gc2_ref[...] + xg
    pg = pg_ref[...]
    pgx = pg[0, 0] + apx
    pgy = pg[0, 1] + apy
    pgz = pg[0, 2] + apz
    x = x_ref[...]
    xin_i = _ln(x, lng_ref[...], lnb_ref[...])
    xg_ln = _ln(xg_new, lng_ref[...], lnb_ref[...])
    basej = jnp.dot(xg_ln, wj_ref[...], preferred_element_type=f32)
    dx = px_ref[...] - pgx
    dy = py_ref[...] - pgy
    dz = pz_ref[...] - pgz
    dist = jnp.sqrt(dx * dx + dy * dy + dz * dz)
    h1 = _silu(jnp.dot(xin_i, wi_ref[...], preferred_element_type=f32)
               + basej + dist * wd_ref[...] + b1_ref[...])
    h2 = _silu(jnp.dot(h1, w2_ref[...], preferred_element_type=f32)
               + b2_ref[...])
    scale = cs_ref[0, 0] / jnp.clip(dist, 1e-8, None)
    pw = jnp.dot(_silu(jnp.dot(h2, p1_ref[...], preferred_element_type=f32)
                       + d1_ref[...]),
                 p2_ref[...], preferred_element_type=f32) * scale
    u = _silu(jnp.dot(x, u1x_ref[...], preferred_element_type=f32)
              + jnp.dot(h2, u1a_ref[...], preferred_element_type=f32)
              + c1_ref[...])
    xo_ref[...] = jnp.dot(u, u2_ref[...], preferred_element_type=f32) \
        + c2_ref[...] + x
    pxo_ref[...] = px_ref[...] + dx * pw
    pyo_ref[...] = py_ref[...] + dy * pw
    pzo_ref[...] = pz_ref[...] + dz * pw

    @pl.when(i == 0)
    def _():
        xgo_ref[...] = xg_new
        pgo_ref[...] = jnp.concatenate(
            [jnp.full((1, 1), pgx, f32), jnp.full((1, 1), pgy, f32),
             jnp.full((1, 1), pgz, f32)], axis=1)


def _g2a_call(x, px, py, pz, xg, pg, hs, psx, psy, psz, pg_upd, p,
              blk=2000):
    n = x.shape[0]
    nb8 = hs.shape[0]
    full = lambda shp: pl.BlockSpec(shp, lambda i: tuple(0 for _ in shp))
    nf = pl.BlockSpec((blk, F), lambda i: (i, 0))
    n1 = pl.BlockSpec((blk, 1), lambda i: (i, 0))
    body = functools.partial(_g2a_body, n_atoms=float(n))
    return pl.pallas_call(
        body,
        grid=(n // blk,),
        in_specs=[nf, n1, n1, n1, full((1, F)), full((1, 3)),
                  full((nb8, F)), full((nb8, 1)), full((nb8, 1)),
                  full((nb8, 1)),
                  full((F, F)), full((F, F)), full((1, F)), full((F, F)),
                  full((1, F)),
                  full((1, F)), full((1, F)), full((F, F)), full((F, F)),
                  full((1, F)), full((1, F)), full((F, F)), full((1, F)),
                  full((F, F)), full((1, F)), full((F, 1)), full((1, 1)),
                  full((F, F)), full((F, F)), full((1, F)), full((F, F)),
                  full((1, F))],
        out_specs=[nf, n1, n1, n1,
                   pl.BlockSpec((1, F), lambda i: (0, 0)),
                   pl.BlockSpec((1, 3), lambda i: (0, 0))],
        out_shape=[jax.ShapeDtypeStruct((n, F), f32),
                   jax.ShapeDtypeStruct((n, 1), f32),
                   jax.ShapeDtypeStruct((n, 1), f32),
                   jax.ShapeDtypeStruct((n, 1), f32),
                   jax.ShapeDtypeStruct((1, F), f32),
                   jax.ShapeDtypeStruct((1, 3), f32)],
    )(x, px, py, pz, xg, pg, hs, psx, psy, psz,
      pg_upd['U1'][:F], pg_upd['U1'][F:], pg_upd['c1'].reshape(1, F),
      pg_upd['U2'], pg_upd['c2'].reshape(1, F),
      p['ln_g'].reshape(1, F), p['ln_b'].reshape(1, F),
      p['W1'][:F], p['W1'][F:2 * F], p['W1'][2 * F:].reshape(1, F),
      p['b1'].reshape(1, F), p['W2'], p['b2'].reshape(1, F),
      p['P1'], p['d1'].reshape(1, F), p['P2'],
      p['coors_scale'].reshape(1, 1),
      p['U1'][:F], p['U1'][F:], p['c1'].reshape(1, F),
      p['U2'], p['c2'].reshape(1, F))


def kernel(x_atom, pos_atom, x_global_node, pos_global_node,
           edge_index_atom_atom, edge_index_atom_global_node,
           edge_index_global_node_atom, params):
    n = x_atom.shape[0]
    e = edge_index_atom_atom.shape[1]
    row = edge_index_atom_atom[0]
    col = edge_index_atom_atom[1]
    ew = e // NW
    col3 = col.reshape(NW, ew // C, C)

    x = x_atom
    px = pos_atom[:, 0:1]
    py = pos_atom[:, 1:2]
    pz = pos_atom[:, 2:3]
    xg = x_global_node
    pg = pos_global_node

    for l in range(len(params['layers'])):
        pa = params['layers'][l]['a2a']
        pag = params['layers'][l]['a2g']
        pga = params['layers'][l]['g2a']

        xi, xj = _pre_call(x, pa['W1'][:F], pa['W1'][F:2 * F])
        gi, gj, dx, dy, dz = _sc_gather_call(
            xi, xj, px.reshape(n), py.reshape(n), pz.reshape(n), col, row)
        h, pm4 = _edge_call(
            gi, gj, dx.reshape(e, 1), dy.reshape(e, 1), dz.reshape(e, 1),
            pa['W1'][2 * F:].reshape(1, F), pa['b1'].reshape(1, F),
            pa['W2'], pa['b2'].reshape(1, F),
            pa['Wa'], pa['ba'].reshape(1, 1),
            pa['P1'], pa['d1'].reshape(1, F), pa['P2'])
        aggh, cnt, ax, ay, az = _sc_scatter_call(
            h, pm4[:, 0], pm4[:, 1], pm4[:, 2], col3, n)
        x, px, py, pz = _upd_call(
            x, aggh, cnt.reshape(NC, n, 1), ax.reshape(NC, n, 1),
            ay.reshape(NC, n, 1), az.reshape(NC, n, 1), px, py, pz,
            pa['U1'][:F], pa['U1'][F:], pa['c1'].reshape(1, F),
            pa['U2'], pa['c2'].reshape(1, F))

        hs, psx, psy, psz = _a2g_call(x, px, py, pz, xg, pg, pag)
        nb = hs.shape[0]
        hs = hs.reshape(nb, F)
        psx = psx.reshape(nb, 1)
        psy = psy.reshape(nb, 1)
        psz = psz.reshape(nb, 1)
        pad = (-nb) % 8
        if pad:
            hs = jnp.pad(hs, ((0, pad), (0, 0)))
            psx = jnp.pad(psx, ((0, pad), (0, 0)))
            psy = jnp.pad(psy, ((0, pad), (0, 0)))
            psz = jnp.pad(psz, ((0, pad), (0, 0)))

        x, px, py, pz, xg, pg = _g2a_call(
            x, px, py, pz, xg, pg, hs, psx, psy, psz, pag, pga)

    pos_atom_out = jnp.concatenate([px, py, pz], axis=1)
    return x, pos_atom_out, xg, pg


# speedup vs baseline: 3.6552x; 1.0111x over previous
import functools

import jax
import jax.numpy as jnp
from jax import lax
from jax.experimental import pallas as pl
from jax.experimental.pallas import tpu as pltpu
from jax.experimental.pallas import tpu_sc as plsc

F = 128
NC = 2
NS = 16
NW = NC * NS
C = 80

f32 = jnp.float32


def _silu(x):
    return x * jax.nn.sigmoid(x)


def _ln(x, g, b, eps=1e-5):
    m = jnp.mean(x, axis=-1, keepdims=True)
    v = jnp.mean((x - m) ** 2, axis=-1, keepdims=True)
    return (x - m) / jnp.sqrt(v + eps) * g + b


def _pre_body(x_ref, wi_ref, wj_ref, xi_ref, xj_ref):
    x = x_ref[...]
    xi_ref[...] = jnp.dot(x, wi_ref[...], preferred_element_type=f32)
    xj_ref[...] = jnp.dot(x, wj_ref[...], preferred_element_type=f32)


def _pre_call(x, wi, wj, blk=2000):
    n = x.shape[0]
    return pl.pallas_call(
        _pre_body,
        grid=(n // blk,),
        in_specs=[pl.BlockSpec((blk, F), lambda i: (i, 0)),
                  pl.BlockSpec((F, F), lambda i: (0, 0)),
                  pl.BlockSpec((F, F), lambda i: (0, 0))],
        out_specs=[pl.BlockSpec((blk, F), lambda i: (i, 0))] * 2,
        out_shape=[jax.ShapeDtypeStruct((n, F), f32)] * 2,
    )(x, wi, wj)


SCE = 80


def _sc_gather_call(xi, xj, px, py, pz, col, row):
    e = col.shape[0]
    n = xi.shape[0]
    ew = e // NW
    nsc = ew // SCE
    mesh = plsc.VectorSubcoreMesh(core_axis_name="c", subcore_axis_name="s")

    @functools.partial(
        pl.kernel,
        out_type=[jax.ShapeDtypeStruct((e, F), f32),
                  jax.ShapeDtypeStruct((e, F), f32),
                  jax.ShapeDtypeStruct((e,), f32),
                  jax.ShapeDtypeStruct((e,), f32),
                  jax.ShapeDtypeStruct((e,), f32)],
        mesh=mesh,
        scratch_types=[pltpu.VMEM((SCE,), jnp.int32),
                       pltpu.VMEM((SCE,), jnp.int32),
                       pltpu.VMEM((SCE,), jnp.int32),
                       pltpu.VMEM((SCE,), jnp.int32),
                       pltpu.VMEM((n,), f32),
                       pltpu.VMEM((n,), f32),
                       pltpu.VMEM((n,), f32),
                       pltpu.VMEM((ew,), f32),
                       pltpu.VMEM((ew,), f32),
                       pltpu.VMEM((ew,), f32),
                       pltpu.VMEM((SCE, F), f32),
                       pltpu.VMEM((SCE, F), f32),
                       pltpu.VMEM((SCE, F), f32),
                       pltpu.VMEM((SCE, F), f32),
                       pltpu.SemaphoreType.DMA,
                       pltpu.SemaphoreType.DMA,
                       pltpu.SemaphoreType.DMA,
                       pltpu.SemaphoreType.DMA,
                       pltpu.SemaphoreType.DMA,
                       pltpu.SemaphoreType.DMA],
        compiler_params=pltpu.CompilerParams(needs_layout_passes=False),
    )
    def k(xi_h, xj_h, px_h, py_h, pz_h, col_h, row_h,
          gi_h, gj_h, dx_h, dy_h, dz_h,
          ica, icb, ira, irb, pxv, pyv, pzv, dbx, dby, dbz,
          obia, obib, obja, objb,
          sia, sib, sga, sgb, swa, swb):
        wid = lax.axis_index("s") * NC + lax.axis_index("c")
        base = wid * ew
        pltpu.sync_copy(px_h, pxv)
        pltpu.sync_copy(py_h, pyv)
        pltpu.sync_copy(pz_h, pzv)

        ic = (ica, icb)
        ir = (ira, irb)
        obi = (obia, obib)
        obj = (obja, objb)
        si = (sia, sib)
        sg = (sga, sgb)
        sw = (swa, swb)

        def idescs(t, p):
            sl = pl.ds(base + t * SCE, SCE)
            return [(col_h.at[sl], ic[p], si[p]),
                    (row_h.at[sl], ir[p], si[p])]

        def istart(t, p):
            for (src, dst, sem) in idescs(t, p):
                pltpu.async_copy(src, dst, sem)

        def iwait(t, p):
            for (src, dst, sem) in idescs(t, p):
                pltpu.make_async_copy(src, dst, sem).wait()

        def gdescs(t, p):
            return [(xi_h.at[ic[p]], obi[p], sg[p]),
                    (xj_h.at[ir[p]], obj[p], sg[p])]

        def gstart(t, p):
            for (src, dst, sem) in gdescs(t, p):
                pltpu.async_copy(src, dst, sem)

        def gwait(t, p):
            for (src, dst, sem) in gdescs(t, p):
                pltpu.make_async_copy(src, dst, sem).wait()

        def wdescs(t, p):
            osl = pl.ds(base + t * SCE, SCE)
            return [(obi[p], gi_h.at[osl], sw[p]),
                    (obj[p], gj_h.at[osl], sw[p])]

        def wstart(t, p):
            for (src, dst, sem) in wdescs(t, p):
                pltpu.async_copy(src, dst, sem)

        def wwait(t, p):
            for (src, dst, sem) in wdescs(t, p):
                pltpu.make_async_copy(src, dst, sem).wait()

        def dircompute(t, p):
            def g16(q, carry):
                sl = pl.ds(q * 16, 16)
                osl = pl.ds(t * SCE + q * 16, 16)
                cv = ic[p][sl]
                rv = ir[p][sl]
                dbx[osl] = (plsc.load_gather(pxv, [cv])
                            - plsc.load_gather(pxv, [rv]))
                dby[osl] = (plsc.load_gather(pyv, [cv])
                            - plsc.load_gather(pyv, [rv]))
                dbz[osl] = (plsc.load_gather(pzv, [cv])
                            - plsc.load_gather(pzv, [rv]))
                return carry

            lax.fori_loop(0, SCE // 16, g16, 0)

        istart(0, 0)
        iwait(0, 0)
        istart(1, 1)
        gstart(0, 0)
        dircompute(0, 0)
        gwait(0, 0)
        wstart(0, 0)

        def pair(tt, carry):
            t1 = 2 * tt + 1
            iwait(t1, 1)

            @pl.when(t1 >= 2)
            def _():
                wwait(t1 - 2, 1)

            gstart(t1, 1)
            istart(t1 + 1, 0)
            dircompute(t1, 1)
            gwait(t1, 1)
            wstart(t1, 1)

            t2 = 2 * tt + 2
            iwait(t2, 0)
            wwait(t2 - 2, 0)
            gstart(t2, 0)

            @pl.when(t2 + 1 < nsc)
            def _():
                istart(t2 + 1, 1)

            dircompute(t2, 0)
            gwait(t2, 0)
            wstart(t2, 0)
            return carry

        lax.fori_loop(0, (nsc - 1) // 2, pair, 0)
        wwait(nsc - 2, 1)
        wwait(nsc - 1, 0)
        pltpu.sync_copy(dbx, dx_h.at[pl.ds(base, ew)])
        pltpu.sync_copy(dby, dy_h.at[pl.ds(base, ew)])
        pltpu.sync_copy(dbz, dz_h.at[pl.ds(base, ew)])

    return k(xi, xj, px, py, pz, col, row)


def _sc_scatter_call(h, pmx, pmy, pmz, col3, n):
    e = h.shape[0]
    ew = e // NW
    nch = col3.shape[1]
    nseg = (n // NS) // 8 * 8
    ntail = n - NS * nseg
    sb = max(d for d in range(8, 49, 8) if nseg % d == 0)
    reps = nseg // sb
    mesh = plsc.VectorSubcoreMesh(core_axis_name="c", subcore_axis_name="s")

    @functools.partial(
        pl.kernel,
        out_type=[jax.ShapeDtypeStruct((NC, n, F), f32),
                  jax.ShapeDtypeStruct((NC * n,), f32),
                  jax.ShapeDtypeStruct((NC * n,), f32),
                  jax.ShapeDtypeStruct((NC * n,), f32),
                  jax.ShapeDtypeStruct((NC * n,), f32)],
        mesh=mesh,
        scratch_types=[pltpu.VMEM((nch, C), jnp.int32),
                       pltpu.VMEM((C,), f32),
                       pltpu.VMEM((C,), f32),
                       pltpu.VMEM((C,), f32),
                       pltpu.VMEM((C,), f32),
                       pltpu.VMEM((C,), f32),
                       pltpu.VMEM((C,), f32),
                       pltpu.VMEM((C,), f32),
                       pltpu.VMEM((C, F), f32),
                       pltpu.VMEM((C, F), f32),
                       pltpu.VMEM((sb, F), f32),
                       pltpu.VMEM((nseg,), f32),
                       pltpu.SemaphoreType.DMA,
                       pltpu.SemaphoreType.DMA,
                       pltpu.SemaphoreType.DMA,
                       pltpu.SemaphoreType.DMA,
                       pltpu.SemaphoreType.DMA,
                       pltpu.VMEM_SHARED((n, F), f32),
                       pltpu.VMEM_SHARED((n,), f32),
                       pltpu.VMEM_SHARED((n,), f32),
                       pltpu.VMEM_SHARED((n,), f32),
                       pltpu.VMEM_SHARED((n,), f32)],
        compiler_params=pltpu.CompilerParams(needs_layout_passes=False),
    )
    def k(h_h, pmx_h, pmy_h, pmz_h, col3_h,
          aggh_h, cnt_h, ax_h, ay_h, az_h,
          idxb, pxa, pxb2, pya, pyb2, pza, pzb2, onesb, hba, hbb, sbuf, pbuf,
          sha, shb, saa, sab, sp4,
          acch, accc, accx, accy, accz):
        c = lax.axis_index("c")
        s = lax.axis_index("s")
        wid = s * NC + c
        base = wid * ew
        hb = (hba, hbb)
        pxs = (pxa, pxb2)
        pys = (pya, pyb2)
        pzs = (pza, pzb2)
        sh = (sha, shb)
        sa = (saa, sab)

        def zb(j, carry):
            sbuf[j // 8, pl.ds((j % 8) * 16, 16)] = jnp.zeros((16,), f32)
            return carry

        lax.fori_loop(0, sb * 8, zb, 0)

        def zp(j, carry):
            pbuf[pl.ds(j * 16, 16)] = jnp.zeros((16,), f32)
            return carry

        lax.fori_loop(0, nseg // 16, zp, 0)

        def ob(j, carry):
            onesb[pl.ds(j * 16, 16)] = jnp.ones((16,), f32)
            return carry

        lax.fori_loop(0, C // 16, ob, 0)

        pltpu.sync_copy(col3_h.at[wid], idxb)

        for r in range(reps):
            pltpu.sync_copy(sbuf, acch.at[pl.ds(s * nseg + r * sb, sb)])
        segsl = pl.ds(s * nseg, nseg)
        pltpu.sync_copy(pbuf, accc.at[segsl])
        pltpu.sync_copy(pbuf, accx.at[segsl])
        pltpu.sync_copy(pbuf, accy.at[segsl])
        pltpu.sync_copy(pbuf, accz.at[segsl])

        @pl.when(s == 0)
        def _():
            tl = pl.ds(NS * nseg, ntail)
            tb = pl.ds(0, ntail)
            pltpu.sync_copy(sbuf.at[tb], acch.at[tl])
            pltpu.sync_copy(pbuf.at[tb], accc.at[tl])
            pltpu.sync_copy(pbuf.at[tb], accx.at[tl])
            pltpu.sync_copy(pbuf.at[tb], accy.at[tl])
            pltpu.sync_copy(pbuf.at[tb], accz.at[tl])

        plsc.subcore_barrier()

        def hdescs(t, p):
            sl = pl.ds(base + t * C, C)
            return [(h_h.at[sl], hb[p], sh[p]),
                    (pmx_h.at[sl], pxs[p], sh[p]),
                    (pmy_h.at[sl], pys[p], sh[p]),
                    (pmz_h.at[sl], pzs[p], sh[p])]

        def hstart(t, p):
            for (src, dst, sem) in hdescs(t, p):
                pltpu.async_copy(src, dst, sem)

        def hwait(t, p):
            for (src, dst, sem) in hdescs(t, p):
                pltpu.make_async_copy(src, dst, sem).wait()

        def adescs(t, p):
            isl = idxb.at[t]
            return [(hb[p], acch.at[isl], sa[p]),
                    (pxs[p], accx.at[isl], sa[p]),
                    (pys[p], accy.at[isl], sa[p]),
                    (pzs[p], accz.at[isl], sa[p])]

        def astart(t, p):
            for (src, dst, sem) in adescs(t, p):
                pltpu.async_copy(src, dst, sem, add=True)
            pltpu.async_copy(onesb, accc.at[idxb.at[t]], sp4, add=True)

        def adrain(t, p):
            for (src, dst, sem) in adescs(t, p):
                pltpu.make_async_copy(src, dst, sem).wait()

        hstart(0, 0)
        hwait(0, 0)
        hstart(1, 1)
        astart(0, 0)

        def pair(tt, carry):
            t1 = 2 * tt + 1
            hwait(t1, 1)
            adrain(t1 - 1, 0)
            hstart(t1 + 1, 0)
            astart(t1, 1)
            t2 = 2 * tt + 2
            hwait(t2, 0)
            adrain(t2 - 1, 1)

            @pl.when(t2 + 1 < nch)
            def _():
                hstart(t2 + 1, 1)

            astart(t2, 0)
            return carry

        lax.fori_loop(0, (nch - 1) // 2, pair, 0)
        adrain(nch - 1, 0)

        def p4drain(t, carry):
            pltpu.make_async_copy(onesb, accc.at[idxb.at[t]], sp4).wait()
            return carry

        lax.fori_loop(0, nch, p4drain, 0)
        plsc.subcore_barrier()

        for r in range(reps):
            ksl = pl.ds(s * nseg + r * sb, sb)
            pltpu.sync_copy(acch.at[ksl], sbuf)
            pltpu.sync_copy(sbuf, aggh_h.at[c, ksl])

        def dump_plane(acc, out_h):
            pltpu.sync_copy(acc.at[segsl], pbuf)
            pltpu.sync_copy(pbuf, out_h.at[pl.ds(c * n + s * nseg, nseg)])

        dump_plane(accc, cnt_h)
        dump_plane(accx, ax_h)
        dump_plane(accy, ay_h)
        dump_plane(accz, az_h)

        @pl.when(s == 0)
        def _():
            tl = pl.ds(NS * nseg, ntail)
            tb = pl.ds(0, ntail)
            pltpu.sync_copy(acch.at[tl], sbuf.at[tb])
            pltpu.sync_copy(sbuf.at[tb], aggh_h.at[c, tl])

            def dump_ptail(acc, out_h):
                pltpu.sync_copy(acc.at[tl], pbuf.at[tb])
                pltpu.sync_copy(pbuf.at[tb],
                                out_h.at[pl.ds(c * n + NS * nseg, ntail)])

            dump_ptail(accc, cnt_h)
            dump_ptail(accx, ax_h)
            dump_ptail(accy, ay_h)
            dump_ptail(accz, az_h)

    return k(h, pmx, pmy, pmz, col3)


def _edge_body(gi_ref, gj_ref, dx_ref, dy_ref, dz_ref,
               wd_ref, b1_ref, w2_ref, b2_ref, wa_ref, ba_ref,
               p1_ref, d1_ref, p2_ref,
               h_ref, pm4_ref):
    dx = dx_ref[...]
    dy = dy_ref[...]
    dz = dz_ref[...]
    dist = jnp.sqrt(dx * dx + dy * dy + dz * dz)
    pre = gi_ref[...] + gj_ref[...] + dist * wd_ref[...] + b1_ref[...]
    h1 = _silu(pre)
    h2 = _silu(jnp.dot(h1, w2_ref[...], preferred_element_type=f32) + b2_ref[...])
    att = jax.nn.sigmoid(jnp.dot(h2, wa_ref[...], preferred_element_type=f32)
                         + ba_ref[...])
    hh = att * h2
    pw = jnp.dot(_silu(jnp.dot(hh, p1_ref[...], preferred_element_type=f32)
                       + d1_ref[...]),
                 p2_ref[...], preferred_element_type=f32)
    h_ref[...] = hh
    pm4_ref[...] = jnp.concatenate(
        [dx * pw, dy * pw, dz * pw, jnp.ones_like(pw)], axis=1)


def _edge_call(gi, gj, dx, dy, dz, wd, b1, w2, b2, wa, ba, p1, d1, p2,
               blk=1280):
    e = gi.shape[0]
    full = lambda shp: pl.BlockSpec(shp, lambda i: tuple(0 for _ in shp))
    ef = pl.BlockSpec((blk, F), lambda i: (i, 0))
    e1 = pl.BlockSpec((blk, 1), lambda i: (i, 0))
    e4 = pl.BlockSpec((blk, 4), lambda i: (i, 0))
    return pl.pallas_call(
        _edge_body,
        grid=(e // blk,),
        in_specs=[ef, ef, e1, e1, e1,
                  full((1, F)), full((1, F)), full((F, F)), full((1, F)),
                  full((F, 1)), full((1, 1)),
                  full((F, F)), full((1, F)), full((F, 1))],
        out_specs=[ef, e4],
        out_shape=[jax.ShapeDtypeStruct((e, F), f32),
                   jax.ShapeDtypeStruct((e, 4), f32)],
    )(gi, gj, dx, dy, dz, wd, b1, w2, b2, wa, ba, p1, d1, p2)


def _upd_a2g_body(x_ref, aggh_ref, cnt_ref, ax_ref, ay_ref, az_ref,
                  px_ref, py_ref, pz_ref,
                  u1x_ref, u1a_ref, c1_ref, u2_ref, c2_ref,
                  xg_ref, pg_ref,
                  lng_ref, lnb_ref, wi_ref, wj_ref, wd_ref, b1_ref,
                  w2_ref, b2_ref, p1_ref, d1_ref, p2_ref, cs_ref,
                  xo_ref, pxo_ref, pyo_ref, pzo_ref,
                  hs_ref, psx_ref, psy_ref, psz_ref):
    x = x_ref[...]
    aggh = aggh_ref[...]
    agg = aggh[0] + aggh[1]
    cntp = cnt_ref[...]
    cnt = jnp.clip(cntp[0] + cntp[1], 1.0, None)
    u = _silu(jnp.dot(x, u1x_ref[...], preferred_element_type=f32)
              + jnp.dot(agg, u1a_ref[...], preferred_element_type=f32)
              + c1_ref[...])
    xn = jnp.dot(u, u2_ref[...], preferred_element_type=f32) \
        + c2_ref[...] + x
    axp = ax_ref[...]
    ayp = ay_ref[...]
    azp = az_ref[...]
    pxn = px_ref[...] + (axp[0] + axp[1]) / cnt
    pyn = py_ref[...] + (ayp[0] + ayp[1]) / cnt
    pzn = pz_ref[...] + (azp[0] + azp[1]) / cnt
    xo_ref[...] = xn
    pxo_ref[...] = pxn
    pyo_ref[...] = pyn
    pzo_ref[...] = pzn
    xg_ln = _ln(xg_ref[...], lng_ref[...], lnb_ref[...])
    basei = jnp.dot(xg_ln, wi_ref[...], preferred_element_type=f32)
    xj = _ln(xn, lng_ref[...], lnb_ref[...])
    pg = pg_ref[...]
    dx = pg[0, 0] - pxn
    dy = pg[0, 1] - pyn
    dz = pg[0, 2] - pzn
    dist = jnp.sqrt(dx * dx + dy * dy + dz * dz)
    h1 = _silu(basei + jnp.dot(xj, wj_ref[...], preferred_element_type=f32)
               + dist * wd_ref[...] + b1_ref[...])
    h2 = _silu(jnp.dot(h1, w2_ref[...], preferred_element_type=f32)
               + b2_ref[...])
    scale = cs_ref[0, 0] / jnp.clip(dist, 1e-8, None)
    pw = jnp.dot(_silu(jnp.dot(h2, p1_ref[...], preferred_element_type=f32)
                       + d1_ref[...]),
                 p2_ref[...], preferred_element_type=f32) * scale
    hs_ref[...] = jnp.sum(h2, axis=0, keepdims=True)[None]
    psx_ref[...] = jnp.sum(dx * pw, axis=0, keepdims=True)[None]
    psy_ref[...] = jnp.sum(dy * pw, axis=0, keepdims=True)[None]
    psz_ref[...] = jnp.sum(dz * pw, axis=0, keepdims=True)[None]


def _upd_a2g_call(x, aggh, cnt, ax, ay, az, px, py, pz,
                  u1x, u1a, c1, u2, c2, xg, pg, p, blk=2000):
    n = x.shape[0]
    nb = n // blk
    full = lambda shp: pl.BlockSpec(shp, lambda i: tuple(0 for _ in shp))
    nf = pl.BlockSpec((blk, F), lambda i: (i, 0))
    n1 = pl.BlockSpec((blk, 1), lambda i: (i, 0))
    pf = pl.BlockSpec((NC, blk, F), lambda i: (0, i, 0))
    p1s = pl.BlockSpec((NC, blk, 1), lambda i: (0, i, 0))
    return pl.pallas_call(
        _upd_a2g_body,
        grid=(nb,),
        in_specs=[nf, pf, p1s, p1s, p1s, p1s, n1, n1, n1,
                  full((F, F)), full((F, F)), full((1, F)),
                  full((F, F)), full((1, F)),
                  full((1, F)), full((1, 3)),
                  full((1, F)), full((1, F)), full((F, F)), full((F, F)),
                  full((1, F)), full((1, F)), full((F, F)), full((1, F)),
                  full((F, F)), full((1, F)), full((F, 1)), full((1, 1))],
        out_specs=[nf, n1, n1, n1,
                   pl.BlockSpec((1, 1, F), lambda i: (i, 0, 0)),
                   pl.BlockSpec((1, 1, 1), lambda i: (i, 0, 0)),
                   pl.BlockSpec((1, 1, 1), lambda i: (i, 0, 0)),
                   pl.BlockSpec((1, 1, 1), lambda i: (i, 0, 0))],
        out_shape=[jax.ShapeDtypeStruct((n, F), f32),
                   jax.ShapeDtypeStruct((n, 1), f32),
                   jax.ShapeDtypeStruct((n, 1), f32),
                   jax.ShapeDtypeStruct((n, 1), f32),
                   jax.ShapeDtypeStruct((nb, 1, F), f32),
                   jax.ShapeDtypeStruct((nb, 1, 1), f32),
                   jax.ShapeDtypeStruct((nb, 1, 1), f32),
                   jax.ShapeDtypeStruct((nb, 1, 1), f32)],
    )(x, aggh, cnt, ax, ay, az, px, py, pz, u1x, u1a, c1, u2, c2,
      xg, pg,
      p['ln_g'].reshape(1, F), p['ln_b'].reshape(1, F),
      p['W1'][:F], p['W1'][F:2 * F], p['W1'][2 * F:].reshape(1, F),
      p['b1'].reshape(1, F), p['W2'], p['b2'].reshape(1, F),
      p['P1'], p['d1'].reshape(1, F), p['P2'],
      p['coors_scale'].reshape(1, 1))


def _a2g_body(x_ref, px_ref, py_ref, pz_ref, xg_ref, pg_ref,
              lng_ref, lnb_ref, wi_ref, wj_ref, wd_ref, b1_ref,
              w2_ref, b2_ref, p1_ref, d1_ref, p2_ref, cs_ref,
              hs_ref, psx_ref, psy_ref, psz_ref):
    xg_ln = _ln(xg_ref[...], lng_ref[...], lnb_ref[...])
    basei = jnp.dot(xg_ln, wi_ref[...], preferred_element_type=f32)
    xj = _ln(x_ref[...], lng_ref[...], lnb_ref[...])
    pg = pg_ref[...]
    dx = pg[0, 0] - px_ref[...]
    dy = pg[0, 1] - py_ref[...]
    dz = pg[0, 2] - pz_ref[...]
    dist = jnp.sqrt(dx * dx + dy * dy + dz * dz)
    h1 = _silu(basei + jnp.dot(xj, wj_ref[...], preferred_element_type=f32)
               + dist * wd_ref[...] + b1_ref[...])
    h2 = _silu(jnp.dot(h1, w2_ref[...], preferred_element_type=f32)
               + b2_ref[...])
    scale = cs_ref[0, 0] / jnp.clip(dist, 1e-8, None)
    pw = jnp.dot(_silu(jnp.dot(h2, p1_ref[...], preferred_element_type=f32)
                       + d1_ref[...]),
                 p2_ref[...], preferred_element_type=f32) * scale
    hs_ref[...] = jnp.sum(h2, axis=0, keepdims=True)[None]
    psx_ref[...] = jnp.sum(dx * pw, axis=0, keepdims=True)[None]
    psy_ref[...] = jnp.sum(dy * pw, axis=0, keepdims=True)[None]
    psz_ref[...] = jnp.sum(dz * pw, axis=0, keepdims=True)[None]


def _a2g_call(x, px, py, pz, xg, pg, p, blk=2000):
    n = x.shape[0]
    nb = n // blk
    full = lambda shp: pl.BlockSpec(shp, lambda i: tuple(0 for _ in shp))
    nf = pl.BlockSpec((blk, F), lambda i: (i, 0))
    n1 = pl.BlockSpec((blk, 1), lambda i: (i, 0))
    return pl.pallas_call(
        _a2g_body,
        grid=(nb,),
        in_specs=[nf, n1, n1, n1, full((1, F)), full((1, 3)),
                  full((1, F)), full((1, F)), full((F, F)), full((F, F)),
                  full((1, F)), full((1, F)), full((F, F)), full((1, F)),
                  full((F, F)), full((1, F)), full((F, 1)), full((1, 1))],
        out_specs=[pl.BlockSpec((1, 1, F), lambda i: (i, 0, 0)),
                   pl.BlockSpec((1, 1, 1), lambda i: (i, 0, 0)),
                   pl.BlockSpec((1, 1, 1), lambda i: (i, 0, 0)),
                   pl.BlockSpec((1, 1, 1), lambda i: (i, 0, 0))],
        out_shape=[jax.ShapeDtypeStruct((nb, 1, F), f32),
                   jax.ShapeDtypeStruct((nb, 1, 1), f32),
                   jax.ShapeDtypeStruct((nb, 1, 1), f32),
                   jax.ShapeDtypeStruct((nb, 1, 1), f32)],
    )(x, px, py, pz, xg, pg,
      p['ln_g'].reshape(1, F), p['ln_b'].reshape(1, F),
      p['W1'][:F], p['W1'][F:2 * F], p['W1'][2 * F:].reshape(1, F),
      p['b1'].reshape(1, F), p['W2'], p['b2'].reshape(1, F),
      p['P1'], p['d1'].reshape(1, F), p['P2'],
      p['coors_scale'].reshape(1, 1))


def _g2a_body(x_ref, px_ref, py_ref, pz_ref, xg_ref, pg_ref,
              hs_ref, psx_ref, psy_ref, psz_ref,
              gu1x_ref, gu1a_ref, gc1_ref, gu2_ref, gc2_ref,
              lng_ref, lnb_ref, wi_ref, wj_ref, wd_ref, b1_ref,
              w2_ref, b2_ref, p1_ref, d1_ref, p2_ref, cs_ref,
              u1x_ref, u1a_ref, c1_ref, u2_ref, c2_ref,
              win_ref, wjn_ref,
              xo_ref, pxo_ref, pyo_ref, pzo_ref, xi_ref, xj_ref,
              xgo_ref, pgo_ref, *, n_atoms):
    i = pl.program_id(0)
    xg = xg_ref[...]
    agg_g = jnp.sum(hs_ref[...], axis=0, keepdims=True)
    inv = 1.0 / n_atoms
    apx = jnp.sum(psx_ref[...]) * inv
    apy = jnp.sum(psy_ref[...]) * inv
    apz = jnp.sum(psz_ref[...]) * inv
    ug = _silu(jnp.dot(xg, gu1x_ref[...], preferred_element_type=f32)
               + jnp.dot(agg_g, gu1a_ref[...], preferred_element_type=f32)
               + gc1_ref[...])
    xg_new = jnp.dot(ug, gu2_ref[...], preferred_element_type=f32) \
        + gc2_ref[...] + xg
    pg = pg_ref[...]
    pgx = pg[0, 0] + apx
    pgy = pg[0, 1] + apy
    pgz = pg[0, 2] + apz
    x = x_ref[...]
    xin_i = _ln(x, lng_ref[...], lnb_ref[...])
    xg_ln = _ln(xg_new, lng_ref[...], lnb_ref[...])
    basej = jnp.dot(xg_ln, wj_ref[...], preferred_element_type=f32)
    dx = px_ref[...] - pgx
    dy = py_ref[...] - pgy
    dz = pz_ref[...] - pgz
    dist = jnp.sqrt(dx * dx + dy * dy + dz * dz)
    h1 = _silu(jnp.dot(xin_i, wi_ref[...], preferred_element_type=f32)
               + basej + dist * wd_ref[...] + b1_ref[...])
    h2 = _silu(jnp.dot(h1, w2_ref[...], preferred_element_type=f32)
               + b2_ref[...])
    scale = cs_ref[0, 0] / jnp.clip(dist, 1e-8, None)
    pw = jnp.dot(_silu(jnp.dot(h2, p1_ref[...], preferred_element_type=f32)
                       + d1_ref[...]),
                 p2_ref[...], preferred_element_type=f32) * scale
    u = _silu(jnp.dot(x, u1x_ref[...], preferred_element_type=f32)
              + jnp.dot(h2, u1a_ref[...], preferred_element_type=f32)
              + c1_ref[...])
    xn = jnp.dot(u, u2_ref[...], preferred_element_type=f32) \
        + c2_ref[...] + x
    xo_ref[...] = xn
    pxo_ref[...] = px_ref[...] + dx * pw
    pyo_ref[...] = py_ref[...] + dy * pw
    pzo_ref[...] = pz_ref[...] + dz * pw
    xi_ref[...] = jnp.dot(xn, win_ref[...], preferred_element_type=f32)
    xj_ref[...] = jnp.dot(xn, wjn_ref[...], preferred_element_type=f32)

    @pl.when(i == 0)
    def _():
        xgo_ref[...] = xg_new
        pgo_ref[...] = jnp.concatenate(
            [jnp.full((1, 1), pgx, f32), jnp.full((1, 1), pgy, f32),
             jnp.full((1, 1), pgz, f32)], axis=1)


def _g2a_call(x, px, py, pz, xg, pg, hs, psx, psy, psz, pg_upd, p,
              wi_next, wj_next, blk=2000):
    n = x.shape[0]
    nb8 = hs.shape[0]
    full = lambda shp: pl.BlockSpec(shp, lambda i: tuple(0 for _ in shp))
    nf = pl.BlockSpec((blk, F), lambda i: (i, 0))
    n1 = pl.BlockSpec((blk, 1), lambda i: (i, 0))
    body = functools.partial(_g2a_body, n_atoms=float(n))
    return pl.pallas_call(
        body,
        grid=(n // blk,),
        in_specs=[nf, n1, n1, n1, full((1, F)), full((1, 3)),
                  full((nb8, F)), full((nb8, 1)), full((nb8, 1)),
                  full((nb8, 1)),
                  full((F, F)), full((F, F)), full((1, F)), full((F, F)),
                  full((1, F)),
                  full((1, F)), full((1, F)), full((F, F)), full((F, F)),
                  full((1, F)), full((1, F)), full((F, F)), full((1, F)),
                  full((F, F)), full((1, F)), full((F, 1)), full((1, 1)),
                  full((F, F)), full((F, F)), full((1, F)), full((F, F)),
                  full((1, F)),
                  full((F, F)), full((F, F))],
        out_specs=[nf, n1, n1, n1, nf, nf,
                   pl.BlockSpec((1, F), lambda i: (0, 0)),
                   pl.BlockSpec((1, 3), lambda i: (0, 0))],
        out_shape=[jax.ShapeDtypeStruct((n, F), f32),
                   jax.ShapeDtypeStruct((n, 1), f32),
                   jax.ShapeDtypeStruct((n, 1), f32),
                   jax.ShapeDtypeStruct((n, 1), f32),
                   jax.ShapeDtypeStruct((n, F), f32),
                   jax.ShapeDtypeStruct((n, F), f32),
                   jax.ShapeDtypeStruct((1, F), f32),
                   jax.ShapeDtypeStruct((1, 3), f32)],
    )(x, px, py, pz, xg, pg, hs, psx, psy, psz,
      pg_upd['U1'][:F], pg_upd['U1'][F:], pg_upd['c1'].reshape(1, F),
      pg_upd['U2'], pg_upd['c2'].reshape(1, F),
      p['ln_g'].reshape(1, F), p['ln_b'].reshape(1, F),
      p['W1'][:F], p['W1'][F:2 * F], p['W1'][2 * F:].reshape(1, F),
      p['b1'].reshape(1, F), p['W2'], p['b2'].reshape(1, F),
      p['P1'], p['d1'].reshape(1, F), p['P2'],
      p['coors_scale'].reshape(1, 1),
      p['U1'][:F], p['U1'][F:], p['c1'].reshape(1, F),
      p['U2'], p['c2'].reshape(1, F),
      wi_next, wj_next)


def kernel(x_atom, pos_atom, x_global_node, pos_global_node,
           edge_index_atom_atom, edge_index_atom_global_node,
           edge_index_global_node_atom, params):
    n = x_atom.shape[0]
    e = edge_index_atom_atom.shape[1]
    row = edge_index_atom_atom[0]
    col = edge_index_atom_atom[1]
    ew = e // NW
    col3 = col.reshape(NW, ew // C, C)

    x = x_atom
    px = pos_atom[:, 0:1]
    py = pos_atom[:, 1:2]
    pz = pos_atom[:, 2:3]
    xg = x_global_node
    pg = pos_global_node

    nlayers = len(params['layers'])
    pa0 = params['layers'][0]['a2a']
    xi, xj = _pre_call(x, pa0['W1'][:F], pa0['W1'][F:2 * F])

    for l in range(nlayers):
        pa = params['layers'][l]['a2a']
        pag = params['layers'][l]['a2g']
        pga = params['layers'][l]['g2a']
        pan = params['layers'][(l + 1) % nlayers]['a2a']

        gi, gj, dx, dy, dz = _sc_gather_call(
            xi, xj, px.reshape(n), py.reshape(n), pz.reshape(n), col, row)
        h, pm4 = _edge_call(
            gi, gj, dx.reshape(e, 1), dy.reshape(e, 1), dz.reshape(e, 1),
            pa['W1'][2 * F:].reshape(1, F), pa['b1'].reshape(1, F),
            pa['W2'], pa['b2'].reshape(1, F),
            pa['Wa'], pa['ba'].reshape(1, 1),
            pa['P1'], pa['d1'].reshape(1, F), pa['P2'])
        aggh, cnt, ax, ay, az = _sc_scatter_call(
            h, pm4[:, 0], pm4[:, 1], pm4[:, 2], col3, n)

        x, px, py, pz, hs, psx, psy, psz = _upd_a2g_call(
            x, aggh, cnt.reshape(NC, n, 1), ax.reshape(NC, n, 1),
            ay.reshape(NC, n, 1), az.reshape(NC, n, 1), px, py, pz,
            pa['U1'][:F], pa['U1'][F:], pa['c1'].reshape(1, F),
            pa['U2'], pa['c2'].reshape(1, F), xg, pg, pag)
        nb = hs.shape[0]
        hs = hs.reshape(nb, F)
        psx = psx.reshape(nb, 1)
        psy = psy.reshape(nb, 1)
        psz = psz.reshape(nb, 1)
        pad = (-nb) % 8
        if pad:
            hs = jnp.pad(hs, ((0, pad), (0, 0)))
            psx = jnp.pad(psx, ((0, pad), (0, 0)))
            psy = jnp.pad(psy, ((0, pad), (0, 0)))
            psz = jnp.pad(psz, ((0, pad), (0, 0)))

        x, px, py, pz, xi, xj, xg, pg = _g2a_call(
            x, px, py, pz, xg, pg, hs, psx, psy, psz, pag, pga,
            pan['W1'][:F], pan['W1'][F:2 * F])

    pos_atom_out = jnp.concatenate([px, py, pz], axis=1)
    return x, pos_atom_out, xg, pg
